# baseline (speedup 1.0000x reference)
import jax
import jax.numpy as jnp
from jax import lax
from jax.experimental import pallas as pl
from jax.experimental.pallas import tpu as pltpu
from jax.experimental.pallas import tpu_sc as plsc

N = 10000
E = 320000
D = 128
DH = 64

NC = 2
NS = 16
L = 16

CH = 128
CPT = 160
NBUF = 4
E_PAD = NS * CPT * CH
N_PAD = 10240
RPT = N_PAD // NS

RB = 2048
GRID = N_PAD // RB

_SC_PARAMS = pltpu.CompilerParams(use_tc_tiling_on_sc=False)


def _make_sc_agg(with_deg: bool):
  mesh = plsc.VectorSubcoreMesh(core_axis_name="c", subcore_axis_name="s")
  out_type = [jax.ShapeDtypeStruct((NC, N_PAD, DH), jnp.float32)]
  if with_deg:
    out_type.append(jax.ShapeDtypeStruct((NC, N_PAD, L), jnp.float32))
  scratch = (
      [pltpu.VMEM((CPT, CH), jnp.int32),
       pltpu.VMEM((CPT, CH), jnp.int32)]
      + [pltpu.VMEM((CH, DH), jnp.float32) for _ in range(NBUF)]
      + [pltpu.VMEM((CH, L), jnp.float32),
         pltpu.VMEM_SHARED((N_PAD, DH), jnp.float32)]
      + ([pltpu.VMEM_SHARED((N_PAD, L), jnp.float32)] if with_deg else [])
      + [pltpu.SemaphoreType.DMA for _ in range(2 * NBUF)]
  )

  def body(feat_hbm, src_hbm, dst_hbm, *rest):
    if with_deg:
      agg_hbm, deg_hbm = rest[0], rest[1]
      rest = rest[2:]
    else:
      agg_hbm = rest[0]
      deg_hbm = deg_sh = None
      rest = rest[1:]
    src_v, dst_v = rest[0], rest[1]
    rows = rest[2:2 + NBUF]
    ones_v = rest[2 + NBUF]
    agg_sh = rest[3 + NBUF]
    if with_deg:
      deg_sh = rest[4 + NBUF]
    sems = rest[-2 * NBUF:]
    gsem, ssem = sems[:NBUF], sems[NBUF:]
    rows_v = rows[0]

    cid = lax.axis_index("c")
    sid = lax.axis_index("s")

    @pl.loop(0, CH)
    def _(r):
      for k in range(DH // L):
        rows_v[r, pl.ds(k * L, L)] = jnp.zeros((L,), jnp.float32)
      ones_v[r, :] = jnp.ones((L,), jnp.float32)

    base = sid * RPT
    for b in range(RPT // CH):
      pltpu.sync_copy(rows_v, agg_sh.at[pl.ds(base + b * CH, CH)])

    pltpu.sync_copy(src_hbm.at[cid, sid], src_v)
    pltpu.sync_copy(dst_hbm.at[sid], dst_v)

    if with_deg:
      @pl.loop(0, CH)
      def _(r):
        ones_v[r, :] = jnp.zeros((L,), jnp.float32)
      for b in range(RPT // CH):
        pltpu.sync_copy(ones_v, deg_sh.at[pl.ds(base + b * CH, CH)])
      @pl.loop(0, CH)
      def _(r):
        ones_v[r, :] = jnp.ones((L,), jnp.float32)

    plsc.subcore_barrier()

    def _deg_turn(b):
      return (cid == 0) if b % 2 == 0 else (cid == 1)

    for b in range(NBUF):
      pltpu.async_copy(feat_hbm.at[src_v.at[b]], rows[b], gsem[b])

    P = CPT // NBUF

    @pl.loop(0, P)
    def _(p):
      j0 = NBUF * p
      for b in range(NBUF):
        pltpu.make_async_copy(feat_hbm.at[src_v.at[j0 + b]],
                              rows[b], gsem[b]).wait()
        pltpu.async_copy(rows[b], agg_sh.at[dst_v.at[j0 + b]], ssem[b],
                         add=True)
        if with_deg:
          @pl.when(_deg_turn(b))
          def _():
            pltpu.async_copy(ones_v, deg_sh.at[dst_v.at[j0 + b]], ssem[b],
                             add=True)
      for b in range(NBUF):
        pltpu.make_async_copy(rows[b], agg_sh.at[dst_v.at[j0 + b]],
                              ssem[b]).wait()
        if with_deg:
          @pl.when(_deg_turn(b))
          def _():
            pltpu.make_async_copy(ones_v, deg_sh.at[dst_v.at[j0 + b]],
                                  ssem[b]).wait()
        @pl.when(p < P - 1)
        def _():
          pltpu.async_copy(feat_hbm.at[src_v.at[j0 + NBUF + b]],
                           rows[b], gsem[b])

    plsc.subcore_barrier()

    pltpu.sync_copy(agg_sh.at[pl.ds(base, RPT)],
                    agg_hbm.at[cid, pl.ds(base, RPT)])
    if with_deg:
      pltpu.sync_copy(deg_sh.at[pl.ds(base, RPT)],
                      deg_hbm.at[cid, pl.ds(base, RPT)])

  return pl.kernel(body, out_type, mesh=mesh, scratch_types=scratch,
                   compiler_params=_SC_PARAMS)


_sc_agg_deg = _make_sc_agg(True)
_sc_agg = _make_sc_agg(False)


def _dotT(a, w):
  return lax.dot_general(a, w, (((1,), (1,)), ((), ())),
                         preferred_element_type=jnp.float32)


def _agg_dotT(agg_ref, deg_ref, wl):
  deg = deg_ref[0, :, 0] + deg_ref[1, :, 0]
  inv = 1.0 / jnp.maximum(deg, 1.0)[:, None]
  return (_dotT(agg_ref[0] * inv, wl[:, :DH]) +
          _dotT(agg_ref[1] * inv, wl[:, DH:]))


def _k12_body(agg_ref, deg_ref, x_ref, wl_ref, wr_ref, b_ref,
              g_ref, be_ref, w2r_ref, hp_ref, hr_ref, acc_ref):
  i = pl.program_id(0)
  h = (_agg_dotT(agg_ref, deg_ref, wl_ref[...]) +
       _dotT(x_ref[...], wr_ref[...]) + b_ref[0])

  @pl.when(i < GRID)
  def _():
    rows = i * RB + lax.broadcasted_iota(jnp.int32, (RB, 1), 0)
    hm = jnp.where(rows < N, h, 0.0)

    @pl.when(i == 0)
    def _():
      acc_ref[...] = jnp.zeros((8, D), jnp.float32)

    acc_ref[0, :] = acc_ref[0, :] + jnp.sum(hm, axis=0)
    acc_ref[1, :] = acc_ref[1, :] + jnp.sum(hm * hm, axis=0)

  @pl.when(i >= GRID)
  def _():
    mu = acc_ref[0, :] / N
    var = acc_ref[1, :] / N - mu * mu
    inv = lax.rsqrt(var + 1e-5)
    hn = (h - mu) * inv * g_ref[0] + be_ref[0]
    hp = jnp.maximum(hn, 0.0)
    hp_ref[...] = hp
    hr_ref[...] = _dotT(hp, w2r_ref[...])


def _k3_body(agg_ref, deg_ref, hr_ref, wl_ref, b_ref, out_ref):
  out_ref[...] = (_agg_dotT(agg_ref, deg_ref, wl_ref[...]) +
                  hr_ref[...] + b_ref[0])


_full = lambda shp: pl.BlockSpec(shp, lambda i: (0,) * len(shp))
_rowblk = pl.BlockSpec((RB, D), lambda i: (i, 0))
_degblk = pl.BlockSpec((NC, RB, L), lambda i: (0, i, 0))
_aggblk = pl.BlockSpec((NC, RB, DH), lambda i: (0, i, 0))

_ph = lambda i: lax.rem(i, GRID)
_phout = lambda i: lax.max(i - GRID, 0)

_k12 = pl.pallas_call(
    _k12_body,
    grid=(2 * GRID,),
    in_specs=[pl.BlockSpec((NC, RB, DH), lambda i: (0, _ph(i), 0)),
              pl.BlockSpec((NC, RB, L), lambda i: (0, _ph(i), 0)),
              pl.BlockSpec((RB, D), lambda i: (_ph(i), 0)),
              _full((D, D)), _full((D, D)), _full((1, D)),
              _full((1, D)), _full((1, D)), _full((D, D))],
    out_specs=[pl.BlockSpec((RB, D), lambda i: (_phout(i), 0)),
               pl.BlockSpec((RB, D), lambda i: (_phout(i), 0))],
    out_shape=[jax.ShapeDtypeStruct((N_PAD, D), jnp.float32),
               jax.ShapeDtypeStruct((N_PAD, D), jnp.float32)],
    scratch_shapes=[pltpu.VMEM((8, D), jnp.float32)],
)

_k3 = pl.pallas_call(
    _k3_body,
    grid=(GRID,),
    in_specs=[_aggblk, _degblk, _rowblk, _full((D, D)), _full((1, D))],
    out_specs=_rowblk,
    out_shape=jax.ShapeDtypeStruct((N, D), jnp.float32),
)


def kernel(x, edge_index, W1_l, b1, W1_r, gamma, beta, W2_l, b2, W2_r):
  src = edge_index[0]
  dst = edge_index[1]
  pad = E_PAD - E
  ar = jnp.arange(pad, dtype=jnp.int32)
  pad_src = (ar * 97) % N
  pad_dst = N + ar % (N_PAD - N)
  src0 = jnp.concatenate([src, pad_src]).reshape(NS, CPT, CH)
  srcp = jnp.stack([2 * src0, 2 * src0 + 1])
  dstp = jnp.concatenate([dst, pad_dst]).reshape(NS, CPT, CH)

  agg1, deg = _sc_agg_deg(x.reshape(2 * N, DH), srcp, dstp)
  h_post, hr = _k12(agg1, deg, x, W1_l, W1_r, b1.reshape(1, D),
                    gamma.reshape(1, D), beta.reshape(1, D), W2_r)
  (agg2,) = _sc_agg(h_post.reshape(2 * N_PAD, DH), srcp, dstp)
  return _k3(agg2, deg, hr, W2_l, b2.reshape(1, D))


# speedup vs baseline: 1.0297x; 1.0297x over previous
import jax
import jax.numpy as jnp
from jax import lax
from jax.experimental import pallas as pl
from jax.experimental.pallas import tpu as pltpu
from jax.experimental.pallas import tpu_sc as plsc

N = 10000
E = 320000
D = 128
DH = 64

NC = 2
NS = 16
L = 16

CH = 128
CPT = 160
NBUF = 4
E_PAD = NS * CPT * CH
N_PAD = 10240
RPT = N_PAD // NS

RB = 2048
GRID = N_PAD // RB

_SC_PARAMS = pltpu.CompilerParams(use_tc_tiling_on_sc=False)


def _make_sc_agg(with_deg: bool):
  mesh = plsc.VectorSubcoreMesh(core_axis_name="c", subcore_axis_name="s")
  out_type = [jax.ShapeDtypeStruct((NC, N_PAD, DH), jnp.float32)]
  if with_deg:
    out_type.append(jax.ShapeDtypeStruct((NC, N_PAD, L), jnp.float32))
  scratch = (
      [pltpu.VMEM((CPT, CH), jnp.int32),
       pltpu.VMEM((CPT, CH), jnp.int32)]
      + [pltpu.VMEM((CH, DH), jnp.float32) for _ in range(NBUF)]
      + [pltpu.VMEM((CH, L), jnp.float32),
         pltpu.VMEM_SHARED((N_PAD, DH), jnp.float32)]
      + ([pltpu.VMEM_SHARED((N_PAD, L), jnp.float32)] if with_deg else [])
      + [pltpu.SemaphoreType.DMA for _ in range(2 * NBUF)]
  )

  def body(feat_hbm, src_hbm, dst_hbm, *rest):
    if with_deg:
      agg_hbm, deg_hbm = rest[0], rest[1]
      rest = rest[2:]
    else:
      agg_hbm = rest[0]
      deg_hbm = deg_sh = None
      rest = rest[1:]
    src_v, dst_v = rest[0], rest[1]
    rows = rest[2:2 + NBUF]
    ones_v = rest[2 + NBUF]
    agg_sh = rest[3 + NBUF]
    if with_deg:
      deg_sh = rest[4 + NBUF]
    sems = rest[-2 * NBUF:]
    gsem, ssem = sems[:NBUF], sems[NBUF:]
    rows_v = rows[0]

    cid = lax.axis_index("c")
    sid = lax.axis_index("s")

    idx_cp_s = pltpu.async_copy(src_hbm.at[cid, sid], src_v, ssem[0])
    idx_cp_d = pltpu.async_copy(dst_hbm.at[sid], dst_v, ssem[1])

    @pl.loop(0, CH)
    def _(r):
      for k in range(DH // L):
        rows_v[r, pl.ds(k * L, L)] = jnp.zeros((L,), jnp.float32)
      ones_v[r, :] = jnp.zeros((L,), jnp.float32)

    base = sid * RPT
    for b in range(RPT // CH):
      pltpu.sync_copy(rows_v, agg_sh.at[pl.ds(base + b * CH, CH)])

    if with_deg:
      for b in range(RPT // CH):
        pltpu.sync_copy(ones_v, deg_sh.at[pl.ds(base + b * CH, CH)])

    @pl.loop(0, CH)
    def _(r):
      ones_v[r, :] = jnp.ones((L,), jnp.float32)

    idx_cp_s.wait()
    idx_cp_d.wait()

    def _deg_turn(b):
      return (cid == 0) if b % 2 == 0 else (cid == 1)

    for b in range(NBUF):
      pltpu.async_copy(feat_hbm.at[src_v.at[b]], rows[b], gsem[b])

    plsc.subcore_barrier()

    P = CPT // NBUF

    @pl.loop(0, P)
    def _(p):
      j0 = NBUF * p
      for b in range(NBUF):
        pltpu.make_async_copy(feat_hbm.at[src_v.at[j0 + b]],
                              rows[b], gsem[b]).wait()
        pltpu.async_copy(rows[b], agg_sh.at[dst_v.at[j0 + b]], ssem[b],
                         add=True)
        if with_deg:
          @pl.when(_deg_turn(b))
          def _():
            pltpu.async_copy(ones_v, deg_sh.at[dst_v.at[j0 + b]], ssem[b],
                             add=True)
      for b in range(NBUF):
        pltpu.make_async_copy(rows[b], agg_sh.at[dst_v.at[j0 + b]],
                              ssem[b]).wait()
        if with_deg:
          @pl.when(_deg_turn(b))
          def _():
            pltpu.make_async_copy(ones_v, deg_sh.at[dst_v.at[j0 + b]],
                                  ssem[b]).wait()
        @pl.when(p < P - 1)
        def _():
          pltpu.async_copy(feat_hbm.at[src_v.at[j0 + NBUF + b]],
                           rows[b], gsem[b])

    plsc.subcore_barrier()

    pltpu.sync_copy(agg_sh.at[pl.ds(base, RPT)],
                    agg_hbm.at[cid, pl.ds(base, RPT)])
    if with_deg:
      pltpu.sync_copy(deg_sh.at[pl.ds(base, RPT)],
                      deg_hbm.at[cid, pl.ds(base, RPT)])

  return pl.kernel(body, out_type, mesh=mesh, scratch_types=scratch,
                   compiler_params=_SC_PARAMS)


_sc_agg_deg = _make_sc_agg(True)
_sc_agg = _make_sc_agg(False)


def _dotT(a, w):
  return lax.dot_general(a, w, (((1,), (1,)), ((), ())),
                         preferred_element_type=jnp.float32)


def _agg_dotT(agg_ref, deg_ref, wl):
  deg = deg_ref[0, :, 0] + deg_ref[1, :, 0]
  inv = 1.0 / jnp.maximum(deg, 1.0)[:, None]
  return (_dotT(agg_ref[0] * inv, wl[:, :DH]) +
          _dotT(agg_ref[1] * inv, wl[:, DH:]))


def _k1_body(agg_ref, deg_ref, x_ref, wl_ref, wr_ref, b_ref,
             h_ref, stats_ref, acc_ref):
  i = pl.program_id(0)
  h = (_agg_dotT(agg_ref, deg_ref, wl_ref[...]) +
       _dotT(x_ref[...], wr_ref[...]) + b_ref[0])
  h_ref[...] = h
  rows = i * RB + lax.broadcasted_iota(jnp.int32, (RB, 1), 0)
  hm = jnp.where(rows < N, h, 0.0)

  @pl.when(i == 0)
  def _():
    acc_ref[...] = jnp.zeros((8, D), jnp.float32)

  acc_ref[0, :] = acc_ref[0, :] + jnp.sum(hm, axis=0)
  acc_ref[1, :] = acc_ref[1, :] + jnp.sum(hm * hm, axis=0)

  @pl.when(i == GRID - 1)
  def _():
    stats_ref[...] = acc_ref[...]


def _k2_body(h_ref, stats_ref, g_ref, be_ref, w2r_ref, hp_ref, hr_ref):
  mu = stats_ref[0, :] / N
  var = stats_ref[1, :] / N - mu * mu
  inv = lax.rsqrt(var + 1e-5)
  hn = (h_ref[...] - mu) * inv * g_ref[0] + be_ref[0]
  hp = jnp.maximum(hn, 0.0)
  hp_ref[...] = hp
  hr_ref[...] = _dotT(hp, w2r_ref[...])


def _k3_body(agg_ref, deg_ref, hr_ref, wl_ref, b_ref, out_ref):
  out_ref[...] = (_agg_dotT(agg_ref, deg_ref, wl_ref[...]) +
                  hr_ref[...] + b_ref[0])


_full = lambda shp: pl.BlockSpec(shp, lambda i: (0,) * len(shp))
_rowblk = pl.BlockSpec((RB, D), lambda i: (i, 0))
_degblk = pl.BlockSpec((NC, RB, L), lambda i: (0, i, 0))
_aggblk = pl.BlockSpec((NC, RB, DH), lambda i: (0, i, 0))

_k1 = pl.pallas_call(
    _k1_body,
    grid=(GRID,),
    in_specs=[_aggblk, _degblk, _rowblk, _full((D, D)), _full((D, D)),
              _full((1, D))],
    out_specs=[_rowblk, _full((8, D))],
    out_shape=[jax.ShapeDtypeStruct((N_PAD, D), jnp.float32),
               jax.ShapeDtypeStruct((8, D), jnp.float32)],
    scratch_shapes=[pltpu.VMEM((8, D), jnp.float32)],
)

_k2 = pl.pallas_call(
    _k2_body,
    grid=(GRID,),
    in_specs=[_rowblk, _full((8, D)), _full((1, D)), _full((1, D)),
              _full((D, D))],
    out_specs=[_rowblk, _rowblk],
    out_shape=[jax.ShapeDtypeStruct((N_PAD, D), jnp.float32),
               jax.ShapeDtypeStruct((N_PAD, D), jnp.float32)],
)

_k3 = pl.pallas_call(
    _k3_body,
    grid=(GRID,),
    in_specs=[_aggblk, _degblk, _rowblk, _full((D, D)), _full((1, D))],
    out_specs=_rowblk,
    out_shape=jax.ShapeDtypeStruct((N, D), jnp.float32),
)


def kernel(x, edge_index, W1_l, b1, W1_r, gamma, beta, W2_l, b2, W2_r):
  src = edge_index[0]
  dst = edge_index[1]
  pad = E_PAD - E
  ar = jnp.arange(pad, dtype=jnp.int32)
  pad_src = (ar * 97) % N
  pad_dst = N + ar % (N_PAD - N)
  src0 = jnp.concatenate([src, pad_src]).reshape(NS, CPT, CH)
  srcp = jnp.stack([2 * src0, 2 * src0 + 1])
  dstp = jnp.concatenate([dst, pad_dst]).reshape(NS, CPT, CH)

  agg1, deg = _sc_agg_deg(x.reshape(2 * N, DH), srcp, dstp)
  h_pre, stats = _k1(agg1, deg, x, W1_l, W1_r, b1.reshape(1, D))
  h_post, hr = _k2(h_pre, stats, gamma.reshape(1, D), beta.reshape(1, D), W2_r)
  (agg2,) = _sc_agg(h_post.reshape(2 * N_PAD, DH), srcp, dstp)
  return _k3(agg2, deg, hr, W2_l, b2.reshape(1, D))
